# baseline (speedup 1.0000x reference)
import jax
import jax.numpy as jnp
from jax import lax
from jax.experimental import pallas as pl
from jax.experimental.pallas import tpu as pltpu

CTX = 1280
DIM = 1024
DEPTH = 2
HEADS = 8
DH = 64
NROT = 53
NPOSE = 6 * NROT
RES = 32
INNER = HEADS * DH
N_DET = 20
N_IMG = 8
PIX = RES * RES

_NL = 17
CHUNK = 320
NCHUNK = CTX // CHUNK


def _lnorm(x, g, b):
    m = jnp.mean(x, axis=-1, keepdims=True)
    v = jnp.mean((x - m) ** 2, axis=-1, keepdims=True)
    return (x - m) / jnp.sqrt(v + 1e-5) * g + b


def _softmax(x):
    m = jnp.max(x, axis=-1, keepdims=True)
    e = jnp.exp(x - m)
    return e / jnp.sum(e, axis=-1, keepdims=True)


def _dot(a, b):
    return jnp.dot(a, b, preferred_element_type=jnp.float32)


def _dot_nt(a, b):
    return lax.dot_general(a, b, (((1,), (1,)), ((), ())),
                           preferred_element_type=jnp.float32)


def _dot_tn(a, b):
    return lax.dot_general(a, b, (((0,), (0,)), ((), ())),
                           preferred_element_type=jnp.float32)


def _bdot(a, b):
    return jnp.dot(a.astype(jnp.bfloat16), b.astype(jnp.bfloat16),
                   preferred_element_type=jnp.float32)


def _bdot_nt(a, b):
    return lax.dot_general(a.astype(jnp.bfloat16), b.astype(jnp.bfloat16),
                           (((1,), (1,)), ((), ())),
                           preferred_element_type=jnp.float32)


def _bdot_tn(a, b):
    return lax.dot_general(a.astype(jnp.bfloat16), b.astype(jnp.bfloat16),
                           (((0,), (0,)), ((), ())),
                           preferred_element_type=jnp.float32)


def _body(last_ref, i0_ref, iy_ref, ix_ref, x_ref, xc_ref,
          cqx_ref, cqy_ref, cvx_ref, cvy_ref, tokw_ref, c0_ref, *rest):
    layers = [rest[i * _NL:(i + 1) * _NL] for i in range(DEPTH)]
    off = DEPTH * _NL
    dpw, dpb, dsw, dsb, dcw, dcb, dew, deb = rest[off:off + 8]
    out9_ref, betas_ref, expr_ref, cam_ref = rest[off + 8:off + 12]
    acc_ref, xb0, xb1, sem0, sem1 = rest[off + 12:off + 17]

    b = pl.program_id(0)
    last_b = last_ref[b]

    bufs = [xb0, xb1]
    sems = [sem0, sem1]

    def dma(c, buf, sem):
        return pltpu.make_async_copy(
            x_ref.at[last_b, pl.ds(c * CHUNK, CHUNK)], buf, sem)

    dma(0, bufs[0], sems[0]).start()
    parts = []
    for c in range(NCHUNK):
        if c + 1 < NCHUNK:
            dma(c + 1, bufs[(c + 1) % 2], sems[(c + 1) % 2]).start()
        dma(c, bufs[c % 2], sems[c % 2]).wait()
        parts.append(jnp.reshape(bufs[c % 2][...], (CHUNK, PIX)))
    xrow = jnp.concatenate(parts, axis=0)
    i0 = i0_ref[...]
    iy = iy_ref[...]
    ix = ix_ref[...]

    iota_c = lax.broadcasted_iota(jnp.int32, (N_DET, 1), 0)
    sel = i0 == b
    first_b = jnp.min(jnp.where(sel, iota_c, N_DET))
    cnt_b = jnp.sum(sel.astype(jnp.int32))

    ohy = (lax.broadcasted_iota(jnp.int32, (N_DET, RES), 1) == iy
           ).astype(jnp.float32)
    ohx = (lax.broadcasted_iota(jnp.int32, (N_DET, RES), 1) == ix
           ).astype(jnp.float32)
    ohp = (lax.broadcasted_iota(jnp.int32, (N_DET, PIX), 1) == iy * RES + ix
           ).astype(jnp.float32)

    q_xy = _dot(ohy, cqx_ref[...]) + _dot(ohx, cqy_ref[...])
    v_xy = _dot(ohy, cvx_ref[...]) + _dot(ohx, cvy_ref[...])
    xcq = xc_ref[...] + q_xy

    ic = lax.broadcasted_iota(jnp.int32, (N_DET, N_DET), 0)
    idd = lax.broadcasted_iota(jnp.int32, (N_DET, N_DET), 1)
    P = ((idd == first_b + ic) & (ic < cnt_b)).astype(jnp.float32)

    tokx = _dot(P, xcq)
    vsel = v_xy * sel.astype(jnp.float32)
    ctx = xrow + _dot_tn(vsel, ohp)

    x = _dot(tokx, tokw_ref[...]) + c0_ref[...]
    maskk = lax.broadcasted_iota(jnp.int32, (1, N_DET), 1) < cnt_b
    bias = jnp.where(maskk, 0.0, -1e9)
    scale = DH ** -0.5

    for (ln1g, ln1b, saq, sak, sav, sao, ln2g, ln2b, caq, cakv, cao,
         ln3g, ln3b, fw1, fb1, fw2, fb2) in layers:
        h = _lnorm(x, ln1g[...], ln1b[...])
        q = _dot(h, saq[...].astype(jnp.float32))
        k = _dot(h, sak[...].astype(jnp.float32))
        v = _dot(h, sav[...].astype(jnp.float32))
        outs = []
        for hh in range(HEADS):
            s = slice(hh * DH, (hh + 1) * DH)
            lg = _dot_nt(q[:, s], k[:, s]) * scale + bias
            outs.append(_dot(_softmax(lg), v[:, s]))
        x = x + _dot(jnp.concatenate(outs, axis=1),
                     sao[...].astype(jnp.float32))

        h = _lnorm(x, ln2g[...], ln2b[...])
        qc = _dot(h, caq[...])
        kvw = cakv[...]
        qws = [_dot_nt(qc[:, hh * DH:(hh + 1) * DH],
                       kvw[:, hh * DH:(hh + 1) * DH])
               for hh in range(HEADS)]
        qwall = jnp.concatenate(qws, axis=0)
        w = _softmax(_dot(qwall, ctx) * scale)
        wctx = _dot_nt(w, ctx)
        outs = [_dot(wctx[hh * N_DET:(hh + 1) * N_DET],
                     kvw[:, INNER + hh * DH:INNER + (hh + 1) * DH])
                for hh in range(HEADS)]
        x = x + _dot(jnp.concatenate(outs, axis=1),
                     cao[...].astype(jnp.float32))

        h = _lnorm(x, ln3g[...], ln3b[...])
        x = x + _dot(jax.nn.relu(_dot(h, fw1[...].astype(jnp.float32))
                                 + fb1[...]),
                     fw2[...].astype(jnp.float32)) + fb2[...]

    contrib = _dot_tn(P, x)

    @pl.when(b == 0)
    def _():
        acc_ref[...] = jnp.zeros_like(acc_ref)

    acc_ref[...] += contrib

    @pl.when(b == N_IMG - 1)
    def _():
        tok = acc_ref[...]
        pose = _dot(tok, dpw[...]) + dpb[...]
        betas_ref[...] = _dot(tok, dsw[...]) + dsb[...]
        cam_ref[...] = _dot(tok, dcw[...]) + dcb[...]
        expr_ref[...] = _dot(tok, dew[...]) + deb[...]

        jr = lax.broadcasted_iota(jnp.int32, (NPOSE, NROT), 0)
        rr = lax.broadcasted_iota(jnp.int32, (NPOSE, NROT), 1)

        def sel_mat(kk):
            return (jr == 6 * rr + kk).astype(jnp.float32)

        a1x = _dot(pose, sel_mat(0))
        a2x = _dot(pose, sel_mat(1))
        a1y = _dot(pose, sel_mat(2))
        a2y = _dot(pose, sel_mat(3))
        a1z = _dot(pose, sel_mat(4))
        a2z = _dot(pose, sel_mat(5))
        n1 = jnp.sqrt(a1x * a1x + a1y * a1y + a1z * a1z)
        b1x, b1y, b1z = a1x / n1, a1y / n1, a1z / n1
        dd = b1x * a2x + b1y * a2y + b1z * a2z
        u2x, u2y, u2z = a2x - dd * b1x, a2y - dd * b1y, a2z - dd * b1z
        n2 = jnp.sqrt(u2x * u2x + u2y * u2y + u2z * u2z)
        b2x, b2y, b2z = u2x / n2, u2y / n2, u2z / n2
        b3x = b1y * b2z - b1z * b2y
        b3y = b1z * b2x - b1x * b2z
        b3z = b1x * b2y - b1y * b2x
        out9_ref[0] = b1x
        out9_ref[1] = b2x
        out9_ref[2] = b3x
        out9_ref[3] = b1y
        out9_ref[4] = b2y
        out9_ref[5] = b3y
        out9_ref[6] = b1z
        out9_ref[7] = b2z
        out9_ref[8] = b3z


def kernel(x_central, x, idx_0, idx_det_b, idx_det_y, idx_det_x, params):
    del idx_det_b
    i0 = idx_0.astype(jnp.int32)
    iy = idx_det_y.astype(jnp.int32)
    ix = idx_det_x.astype(jnp.int32)

    n_det = x.shape[0]
    le = (i0[None, :] <= jnp.arange(N_IMG, dtype=jnp.int32)[:, None])
    eq = (i0[None, :] == jnp.arange(N_IMG, dtype=jnp.int32)[:, None])
    last = jnp.where(eq.any(axis=1), le.sum(axis=1) - 1, 0).astype(jnp.int32)

    p = params

    init_tail = jnp.concatenate(
        [p['init_pose'], p['init_betas'], p['init_cam']])[None, :]
    c0 = (init_tail @ p['tok_w'][CTX:] + p['tok_b'][None, :]
          + p['pos'][0])

    operands = [
        i0[:, None], iy[:, None], ix[:, None],
        x, x_central,
        p['cq_x'], p['cq_y'], p['cv_x'], p['cv_y'],
        p['tok_w'][:CTX], c0,
    ]
    for L in p['layers']:
        operands += [
            L['ln1_g'][None, :], L['ln1_b'][None, :],
            L['sa_q'].astype(jnp.bfloat16), L['sa_k'].astype(jnp.bfloat16),
            L['sa_v'].astype(jnp.bfloat16), L['sa_o'].astype(jnp.bfloat16),
            L['ln2_g'][None, :], L['ln2_b'][None, :],
            L['ca_q'], L['ca_kv'], L['ca_o'],
            L['ln3_g'][None, :], L['ln3_b'][None, :],
            L['ff_w1'].astype(jnp.bfloat16), L['ff_b1'][None, :],
            L['ff_w2'].astype(jnp.bfloat16), L['ff_b2'][None, :],
        ]
    operands += [
        p['decpose_w'], (p['decpose_b'] + p['init_pose'])[None, :],
        p['decshape_w'], (p['decshape_b'] + p['init_betas'])[None, :],
        p['deccam_w'], (p['deccam_b'] + p['init_cam'])[None, :],
        p['decexpr_w'], (p['decexpr_b'] + p['init_expr'])[None, :],
    ]

    def const_spec(a):
        nd = a.ndim
        return pl.BlockSpec(a.shape, lambda b, last_ref, _n=nd: (0,) * _n)

    in_specs = []
    for i, a in enumerate(operands):
        if i == 3:
            in_specs.append(pl.BlockSpec(memory_space=pl.ANY))
        else:
            in_specs.append(const_spec(a))

    out_shapes = [
        jax.ShapeDtypeStruct((9, N_DET, NROT), jnp.float32),
        jax.ShapeDtypeStruct((N_DET, 10), jnp.float32),
        jax.ShapeDtypeStruct((N_DET, 10), jnp.float32),
        jax.ShapeDtypeStruct((N_DET, 3), jnp.float32),
    ]
    out_specs = [const_spec(s) for s in out_shapes]

    grid_spec = pltpu.PrefetchScalarGridSpec(
        num_scalar_prefetch=1,
        grid=(N_IMG,),
        in_specs=in_specs,
        out_specs=out_specs,
        scratch_shapes=[
            pltpu.VMEM((N_DET, DIM), jnp.float32),
            pltpu.VMEM((CHUNK, RES, RES), jnp.float32),
            pltpu.VMEM((CHUNK, RES, RES), jnp.float32),
            pltpu.SemaphoreType.DMA,
            pltpu.SemaphoreType.DMA,
        ],
    )

    out9, betas, expr, cam = pl.pallas_call(
        _body,
        grid_spec=grid_spec,
        out_shape=out_shapes,
        compiler_params=pltpu.CompilerParams(
            dimension_semantics=("arbitrary",),
            vmem_limit_bytes=128 * 1024 * 1024,
        ),
    )(last, *operands)

    rot = jnp.transpose(out9, (1, 2, 0)).reshape(n_det, NROT, 3, 3)
    return (rot[:, :1], rot[:, 1:], betas, expr, cam)


# speedup vs baseline: 2.1862x; 2.1862x over previous
import jax
import jax.numpy as jnp
from jax import lax
from jax.experimental import pallas as pl
from jax.experimental.pallas import tpu as pltpu

CTX = 1280
DIM = 1024
DEPTH = 2
HEADS = 8
DH = 64
NROT = 53
NPOSE = 6 * NROT
RES = 32
INNER = HEADS * DH
N_DET = 20
N_IMG = 8
PIX = RES * RES

_NL = 17


def _lnorm(x, g, b):
    m = jnp.mean(x, axis=-1, keepdims=True)
    v = jnp.mean((x - m) ** 2, axis=-1, keepdims=True)
    return (x - m) / jnp.sqrt(v + 1e-5) * g + b


def _softmax(x):
    m = jnp.max(x, axis=-1, keepdims=True)
    e = jnp.exp(x - m)
    return e / jnp.sum(e, axis=-1, keepdims=True)


def _dot(a, b):
    return jnp.dot(a, b, preferred_element_type=jnp.float32)


def _dot_nt(a, b):
    return lax.dot_general(a, b, (((1,), (1,)), ((), ())),
                           preferred_element_type=jnp.float32)


def _dot_tn(a, b):
    return lax.dot_general(a, b, (((0,), (0,)), ((), ())),
                           preferred_element_type=jnp.float32)


def _body(last_ref, i0_ref, iy_ref, ix_ref, x_ref, xc_ref,
          cqx_ref, cqy_ref, cvx_ref, cvy_ref, tokw_ref, tokb_ref, pos_ref,
          ip_ref, ib_ref, ic_ref, ie_ref, *rest):
    layers = [rest[i * _NL:(i + 1) * _NL] for i in range(DEPTH)]
    off = DEPTH * _NL
    dpw, dpb, dsw, dsb, dcw, dcb, dew, deb = rest[off:off + 8]
    out9_ref, betas_ref, expr_ref, cam_ref = rest[off + 8:off + 12]
    acc_ref = rest[off + 12]

    b = pl.program_id(0)
    i0 = jnp.reshape(i0_ref[...], (N_DET, 1))
    iy = jnp.reshape(iy_ref[...], (N_DET, 1))
    ix = jnp.reshape(ix_ref[...], (N_DET, 1))

    iota_c = lax.broadcasted_iota(jnp.int32, (N_DET, 1), 0)
    sel = i0 == b
    first_b = jnp.min(jnp.where(sel, iota_c, N_DET))
    cnt_b = jnp.sum(sel.astype(jnp.int32))

    ohy = (lax.broadcasted_iota(jnp.int32, (N_DET, RES), 1) == iy
           ).astype(jnp.float32)
    ohx = (lax.broadcasted_iota(jnp.int32, (N_DET, RES), 1) == ix
           ).astype(jnp.float32)
    ohp = (lax.broadcasted_iota(jnp.int32, (N_DET, PIX), 1) == iy * RES + ix
           ).astype(jnp.float32)

    q_xy = _dot(ohy, cqx_ref[...]) + _dot(ohx, cqy_ref[...])
    v_xy = _dot(ohy, cvx_ref[...]) + _dot(ohx, cvy_ref[...])
    xcq = xc_ref[...] + q_xy

    ic2 = lax.broadcasted_iota(jnp.int32, (N_DET, N_DET), 0)
    idd = lax.broadcasted_iota(jnp.int32, (N_DET, N_DET), 1)
    P = ((idd == first_b + ic2) & (ic2 < cnt_b)).astype(jnp.float32)

    tokx = _dot(P, xcq)
    vsel = v_xy * sel.astype(jnp.float32)
    ctx = x_ref[0] + _dot_tn(vsel, ohp)

    tail = jnp.reshape(
        jnp.concatenate([ip_ref[...], ib_ref[...], ic_ref[...]]), (1, 331))
    c0 = (_dot(tail, tokw_ref[CTX:, :]) + tokb_ref[...] + pos_ref[...])

    x = _dot(tokx, tokw_ref[:CTX, :]) + c0
    maskk = lax.broadcasted_iota(jnp.int32, (1, N_DET), 1) < cnt_b
    bias = jnp.where(maskk, 0.0, -1e9)
    scale = DH ** -0.5

    for (ln1g, ln1b, saq, sak, sav, sao, ln2g, ln2b, caq, cakv, cao,
         ln3g, ln3b, fw1, fb1, fw2, fb2) in layers:
        h = _lnorm(x, ln1g[...], ln1b[...])
        q = _dot(h, saq[...].astype(jnp.float32))
        k = _dot(h, sak[...].astype(jnp.float32))
        v = _dot(h, sav[...].astype(jnp.float32))
        outs = []
        for hh in range(HEADS):
            s = slice(hh * DH, (hh + 1) * DH)
            lg = _dot_nt(q[:, s], k[:, s]) * scale + bias
            outs.append(_dot(_softmax(lg), v[:, s]))
        x = x + _dot(jnp.concatenate(outs, axis=1),
                     sao[...].astype(jnp.float32))

        h = _lnorm(x, ln2g[...], ln2b[...])
        qc = _dot(h, caq[...])
        kvw = cakv[...]
        qws = [_dot_nt(qc[:, hh * DH:(hh + 1) * DH],
                       kvw[:, hh * DH:(hh + 1) * DH])
               for hh in range(HEADS)]
        qwall = jnp.concatenate(qws, axis=0)
        w = _softmax(_dot(qwall, ctx) * scale)
        wctx = _dot_nt(w, ctx)
        outs = [_dot(wctx[hh * N_DET:(hh + 1) * N_DET],
                     kvw[:, INNER + hh * DH:INNER + (hh + 1) * DH])
                for hh in range(HEADS)]
        x = x + _dot(jnp.concatenate(outs, axis=1), cao[...])

        h = _lnorm(x, ln3g[...], ln3b[...])
        x = x + _dot(jax.nn.relu(_dot(h, fw1[...].astype(jnp.float32))
                                 + fb1[...]),
                     fw2[...].astype(jnp.float32)) + fb2[...]

    contrib = _dot_tn(P, x)

    @pl.when(b == 0)
    def _():
        acc_ref[...] = jnp.zeros_like(acc_ref)

    acc_ref[...] += contrib

    @pl.when(b == N_IMG - 1)
    def _():
        tok = acc_ref[...]
        pose = _dot(tok, dpw[...]) + (dpb[...] + ip_ref[...])
        betas_ref[...] = _dot(tok, dsw[...]) + (dsb[...] + ib_ref[...])
        cam_ref[...] = _dot(tok, dcw[...]) + (dcb[...] + ic_ref[...])
        expr_ref[...] = _dot(tok, dew[...]) + (deb[...] + ie_ref[...])

        jr = lax.broadcasted_iota(jnp.int32, (NPOSE, NROT), 0)
        rr = lax.broadcasted_iota(jnp.int32, (NPOSE, NROT), 1)

        def sel_mat(kk):
            return (jr == 6 * rr + kk).astype(jnp.float32)

        a1x = _dot(pose, sel_mat(0))
        a2x = _dot(pose, sel_mat(1))
        a1y = _dot(pose, sel_mat(2))
        a2y = _dot(pose, sel_mat(3))
        a1z = _dot(pose, sel_mat(4))
        a2z = _dot(pose, sel_mat(5))
        n1 = jnp.sqrt(a1x * a1x + a1y * a1y + a1z * a1z)
        b1x, b1y, b1z = a1x / n1, a1y / n1, a1z / n1
        dd = b1x * a2x + b1y * a2y + b1z * a2z
        u2x, u2y, u2z = a2x - dd * b1x, a2y - dd * b1y, a2z - dd * b1z
        n2 = jnp.sqrt(u2x * u2x + u2y * u2y + u2z * u2z)
        b2x, b2y, b2z = u2x / n2, u2y / n2, u2z / n2
        b3x = b1y * b2z - b1z * b2y
        b3y = b1z * b2x - b1x * b2z
        b3z = b1x * b2y - b1y * b2x
        out9_ref[0] = b1x
        out9_ref[1] = b2x
        out9_ref[2] = b3x
        out9_ref[3] = b1y
        out9_ref[4] = b2y
        out9_ref[5] = b3y
        out9_ref[6] = b1z
        out9_ref[7] = b2z
        out9_ref[8] = b3z


def kernel(x_central, x, idx_0, idx_det_b, idx_det_y, idx_det_x, params):
    del idx_det_b
    i0 = idx_0.astype(jnp.int32)
    iy = idx_det_y.astype(jnp.int32)
    ix = idx_det_x.astype(jnp.int32)

    n_det = x.shape[0]
    ar = jnp.arange(N_IMG, dtype=jnp.int32)[:, None]
    le = i0[None, :] <= ar
    eq = i0[None, :] == ar
    last = jnp.where(eq.any(axis=1), le.sum(axis=1) - 1, 0).astype(jnp.int32)

    x3 = x.reshape(n_det, CTX, PIX)
    p = params

    operands = [
        i0, iy, ix,
        x3, x_central,
        p['cq_x'], p['cq_y'], p['cv_x'], p['cv_y'],
        p['tok_w'], p['tok_b'], p['pos'][0],
        p['init_pose'], p['init_betas'], p['init_cam'], p['init_expr'],
    ]
    for L in p['layers']:
        operands += [
            L['ln1_g'], L['ln1_b'],
            L['sa_q'].astype(jnp.bfloat16), L['sa_k'].astype(jnp.bfloat16),
            L['sa_v'].astype(jnp.bfloat16), L['sa_o'].astype(jnp.bfloat16),
            L['ln2_g'], L['ln2_b'],
            L['ca_q'], L['ca_kv'], L['ca_o'],
            L['ln3_g'], L['ln3_b'],
            L['ff_w1'].astype(jnp.bfloat16), L['ff_b1'],
            L['ff_w2'].astype(jnp.bfloat16), L['ff_b2'],
        ]
    operands += [
        p['decpose_w'], p['decpose_b'],
        p['decshape_w'], p['decshape_b'],
        p['deccam_w'], p['deccam_b'],
        p['decexpr_w'], p['decexpr_b'],
    ]

    def const_spec(a):
        nd = a.ndim
        return pl.BlockSpec(a.shape, lambda b, last_ref, _n=nd: (0,) * _n)

    in_specs = []
    for i, a in enumerate(operands):
        if i == 3:
            in_specs.append(pl.BlockSpec(
                (1, CTX, PIX), lambda b, last_ref: (last_ref[b], 0, 0)))
        else:
            in_specs.append(const_spec(a))

    out_shapes = [
        jax.ShapeDtypeStruct((9, N_DET, NROT), jnp.float32),
        jax.ShapeDtypeStruct((N_DET, 10), jnp.float32),
        jax.ShapeDtypeStruct((N_DET, 10), jnp.float32),
        jax.ShapeDtypeStruct((N_DET, 3), jnp.float32),
    ]
    out_specs = [const_spec(s) for s in out_shapes]

    grid_spec = pltpu.PrefetchScalarGridSpec(
        num_scalar_prefetch=1,
        grid=(N_IMG,),
        in_specs=in_specs,
        out_specs=out_specs,
        scratch_shapes=[pltpu.VMEM((N_DET, DIM), jnp.float32)],
    )

    out9, betas, expr, cam = pl.pallas_call(
        _body,
        grid_spec=grid_spec,
        out_shape=out_shapes,
        compiler_params=pltpu.CompilerParams(
            dimension_semantics=("arbitrary",),
            vmem_limit_bytes=128 * 1024 * 1024,
        ),
    )(last, *operands)

    rot = jnp.transpose(out9, (1, 2, 0)).reshape(n_det, NROT, 3, 3)
    return (rot[:, :1], rot[:, 1:], betas, expr, cam)
